# baseline (speedup 1.0000x reference)
import functools

import jax
import jax.numpy as jnp
from jax import lax
from jax.experimental import pallas as pl
from jax.experimental.pallas import tpu as pltpu
from jax.experimental.pallas import tpu_sc as plsc

DIM = 64
BATCH = 16384
NC, NS = 2, 16
NW = NC * NS
B_PER_W = BATCH // NW
L = 16

_mesh = plsc.VectorSubcoreMesh(core_axis_name="c", subcore_axis_name="s")


@functools.partial(
    pl.kernel,
    mesh=_mesh,
    out_type=jax.ShapeDtypeStruct((BATCH, DIM), jnp.float32),
    scratch_types=[
        pltpu.VMEM((B_PER_W,), jnp.int32),
        pltpu.VMEM((8, 8, 128), jnp.float32),
        pltpu.VMEM((B_PER_W, DIM), jnp.float32),
        pltpu.SemaphoreType.DMA,
    ],
    compiler_params=pltpu.CompilerParams(needs_layout_passes=False),
)
def _gather_kernel(tabT_hbm, idx_hbm, out_hbm, idx_v, win_v, rows_v, sem):
    wid = lax.axis_index("s") * NC + lax.axis_index("c")
    base = wid * B_PER_W
    pltpu.sync_copy(idx_hbm.at[pl.ds(base, B_PER_W)], idx_v)

    lanes = lax.iota(jnp.int32, L)

    def per_group(g, _):
        v = idx_v[pl.ds(g * L, L)]
        for l in range(L):
            s = lax.squeeze(lax.slice(v, (l,), (l + 1,)), (0,))
            k = pl.multiple_of((s >> 7) << 7, 128)
            col = jnp.full((L,), s & 127, jnp.int32)
            for r in range(8):
                pltpu.make_async_copy(
                    tabT_hbm.at[pl.ds(8 * r, 8), pl.ds(k, 128)],
                    win_v.at[r], sem,
                ).start()
            pltpu.make_async_copy(
                tabT_hbm.at[pl.ds(0, 64), pl.ds(0, 128)], win_v, sem
            ).wait()
            j = g * L + l
            for q in range(4):
                dvec = 16 * q + lanes
                vals = plsc.load_gather(win_v, [dvec >> 3, dvec & 7, col])
                rows_v[j, pl.ds(16 * q, L)] = vals
        return ()

    lax.fori_loop(0, B_PER_W // L, per_group, ())
    pltpu.sync_copy(rows_v, out_hbm.at[pl.ds(base, B_PER_W)])


def kernel(inputs, table):
    return _gather_kernel(table.T, inputs)
